# baseline (speedup 1.0000x reference)
import functools

import jax
import jax.numpy as jnp
from jax import lax
from jax.experimental import pallas as pl
from jax.experimental.pallas import tpu as pltpu
from jax.experimental.pallas import tpu_sc as plsc

D_MODEL = 64
SCALE = 8.0
NW = 32
GROUP = 128
N_TOTAL = 4096 * 200
PER_W = N_TOTAL // NW
N_GROUPS = PER_W // GROUP

_mesh = plsc.VectorSubcoreMesh(core_axis_name="c", subcore_axis_name="s")


@functools.partial(
    pl.kernel,
    mesh=_mesh,
    out_type=jax.ShapeDtypeStruct((N_TOTAL, D_MODEL), jnp.float32),
    scratch_types=[
        pltpu.VMEM((N_GROUPS, GROUP), jnp.int32),
        pltpu.VMEM((GROUP, D_MODEL), jnp.float32),
        pltpu.SemaphoreType.DMA,
    ],
    compiler_params=pltpu.CompilerParams(use_tc_tiling_on_sc=False),
)
def _emb_lookup(idx_hbm, lut_hbm, out_hbm, idx_v, rows_v, sem):
    c = lax.axis_index("c")
    s = lax.axis_index("s")
    wid = s * 2 + c
    base = wid * PER_W

    pltpu.sync_copy(idx_hbm.at[wid], idx_v)

    def group_body(g, carry):
        pltpu.async_copy(lut_hbm.at[idx_v.at[g]], rows_v, sem).wait()

        def row_body(r, carry2):
            for col in range(D_MODEL // 16):
                sl = pl.ds(col * 16, 16)
                rows_v[r, sl] = rows_v[r, sl] * SCALE
            return carry2

        lax.fori_loop(0, GROUP, row_body, 0, unroll=2)

        pltpu.sync_copy(rows_v, out_hbm.at[pl.ds(base + g * GROUP, GROUP)])
        return carry

    lax.fori_loop(0, N_GROUPS, group_body, 0)


def kernel(x, lut):
    idx = x.reshape(NW, N_GROUPS, GROUP).astype(jnp.int32)
    out = _emb_lookup(idx, lut)
    return out.reshape(4096, 200, D_MODEL)


# speedup vs baseline: 1.1609x; 1.1609x over previous
import functools

import jax
import jax.numpy as jnp
from jax import lax
from jax.experimental import pallas as pl
from jax.experimental.pallas import tpu as pltpu
from jax.experimental.pallas import tpu_sc as plsc

D_MODEL = 64
SCALE = 8.0
NW = 32
GROUP = 128
K = 2
SG = K * GROUP
NSLOT = 4
N_TOTAL = 4096 * 200
PER_W = N_TOTAL // NW
N_GROUPS = PER_W // GROUP
NSG = PER_W // SG
N_OUTER = NSG // NSLOT

_mesh = plsc.VectorSubcoreMesh(core_axis_name="c", subcore_axis_name="s")


@functools.partial(
    pl.kernel,
    mesh=_mesh,
    out_type=jax.ShapeDtypeStruct((N_TOTAL, D_MODEL), jnp.float32),
    scratch_types=[
        pltpu.VMEM((N_GROUPS, GROUP), jnp.int32),
        pltpu.VMEM((NSLOT, SG, D_MODEL), jnp.float32),
        pltpu.SemaphoreType.DMA((NSLOT,)),
        pltpu.SemaphoreType.DMA((NSLOT,)),
    ],
    compiler_params=pltpu.CompilerParams(use_tc_tiling_on_sc=False),
)
def _emb_lookup(idx_hbm, lut_hbm, out_hbm, idx_v, rows_v, gsem, wsem):
    c = lax.axis_index("c")
    s = lax.axis_index("s")
    wid = s * 2 + c
    base = wid * PER_W

    pltpu.sync_copy(idx_hbm.at[wid], idx_v)

    def fire_gathers(slot, sg):
        for j in range(K):
            pltpu.async_copy(
                lut_hbm.at[idx_v.at[sg * K + j]],
                rows_v.at[slot, pl.ds(j * GROUP, GROUP)],
                gsem.at[slot],
            )

    def drain_gathers(slot, sg):
        for j in range(K):
            pltpu.make_async_copy(
                lut_hbm.at[idx_v.at[sg * K + j]],
                rows_v.at[slot, pl.ds(j * GROUP, GROUP)],
                gsem.at[slot],
            ).wait()

    def fire_wb(slot, sg):
        pltpu.async_copy(
            rows_v.at[slot], out_hbm.at[pl.ds(base + sg * SG, SG)], wsem.at[slot]
        )

    def drain_wb(slot, sg):
        pltpu.make_async_copy(
            rows_v.at[slot], out_hbm.at[pl.ds(base + sg * SG, SG)], wsem.at[slot]
        ).wait()

    def scale(slot):
        @plsc.parallel_loop(0, SG, unroll=4)
        def _(r):
            for cj in range(D_MODEL // 16):
                sl = pl.ds(cj * 16, 16)
                rows_v[slot, r, sl] = rows_v[slot, r, sl] * SCALE

    fire_gathers(0, 0)
    fire_gathers(1, 1)

    def outer(t, carry):
        for b in range(NSLOT):
            sg = t * NSLOT + b
            drain_gathers(b, sg)
            scale(b)
            nslot = (b + 2) % NSLOT
            nsg = sg + 2

            @pl.when(nsg >= NSLOT)
            def _():
                drain_wb(nslot, nsg - NSLOT)

            @pl.when(nsg < NSG)
            def _():
                fire_gathers(nslot, nsg)

            fire_wb(b, sg)
        return carry

    lax.fori_loop(0, N_OUTER, outer, 0)

    for b in range(NSLOT - 2, NSLOT):
        drain_wb(b, NSG - NSLOT + b)


def kernel(x, lut):
    idx = x.reshape(NW, N_GROUPS, GROUP).astype(jnp.int32)
    out = _emb_lookup(idx, lut)
    return out.reshape(4096, 200, D_MODEL)
